# Initial kernel scaffold; baseline (speedup 1.0000x reference)
import jax
import jax.numpy as jnp
from jax.experimental import pallas as pl


def kernel(x, adj, W1, b1, W2, b2):
    raise NotImplementedError("write your pallas kernel here")


# speedup vs baseline: 1468.1084x; 1468.1084x over previous
import jax
import jax.numpy as jnp
from jax.experimental import pallas as pl


def _gcn2_kernel(xp_ref, adj_ref, w1t_ref, b1_ref, w2t_ref, b2_ref, out_ref):
    adjv = adj_ref[...]
    n = adjv.shape[0]
    dis = jax.lax.rsqrt(1.0 + jnp.sum(adjv, axis=0, keepdims=True))
    r = jax.lax.broadcasted_iota(jnp.int32, (n, n), 0)
    c = jax.lax.broadcasted_iota(jnp.int32, (n, n), 1)
    ahat = adjv + jnp.where(r == c, 1.0, 0.0)

    w1t = w1t_ref[...]
    w2t = w2t_ref[...]
    f1 = w1t.shape[1]
    f2 = w2t.shape[1]
    nb1 = xp_ref.shape[0] // f1

    def layer(h, wt, f, nb, bias):
        agg = jnp.dot(h * dis, ahat, preferred_element_type=jnp.float32) * dis
        pieces = [
            jnp.dot(wt, agg[i * f:(i + 1) * f, :],
                    preferred_element_type=jnp.float32)
            for i in range(nb)
        ]
        z = jnp.concatenate(pieces, axis=0)
        return jnp.maximum(z + bias, 0.0)

    h1 = layer(xp_ref[...], w1t, f1, nb1, b1_ref[...])
    nb2 = h1.shape[0] // f2
    out_ref[...] = layer(h1, w2t, f2, nb2, b2_ref[...])


def kernel(x, adj, W1, b1, W2, b2):
    bt, n, f = x.shape
    o = W2.shape[1]
    xp = x.transpose(0, 2, 1).reshape(bt * f, n)
    b1c = jnp.tile(b1, bt)[:, None]
    b2c = jnp.tile(b2, bt)[:, None]
    outp = pl.pallas_call(
        _gcn2_kernel,
        out_shape=jax.ShapeDtypeStruct((bt * o, n), jnp.float32),
    )(xp, adj, W1.T, b1c, W2.T, b2c)
    return outp.reshape(bt, o, n).transpose(0, 2, 1)
